# Initial kernel scaffold; baseline (speedup 1.0000x reference)
import jax
import jax.numpy as jnp
from jax.experimental import pallas as pl


def kernel(x, Wr, w1, b1, w2, b2):
    raise NotImplementedError("write your pallas kernel here")


# speedup vs baseline: 1.4313x; 1.4313x over previous
import functools
import math

import jax
import jax.numpy as jnp
from jax import lax
from jax.experimental import pallas as pl
from jax.experimental.pallas import tpu as pltpu
from jax.experimental.pallas import tpu_sc as plsc

_NC = 2
_NS = 16
_NW = _NC * _NS


def _router_body(n_tokens, n_experts, capacity, k_top,
                 x_ref, wr_ref, slots_ref, gates_ref, valid_ref, aux_ref):
    f32 = jnp.float32
    xb = x_ref[...]
    wr = wr_ref[...]
    logits = jnp.dot(xb, wr, preferred_element_type=f32)
    m = jnp.max(logits, axis=1, keepdims=True)
    ex = jnp.exp(logits - m)
    probs = ex / jnp.sum(ex, axis=1, keepdims=True)

    eidx = lax.broadcasted_iota(jnp.int32, (n_tokens, n_experts), 1)
    m1 = jnp.max(probs, axis=1, keepdims=True)
    i1 = jnp.min(jnp.where(probs == m1, eidx, n_experts), axis=1,
                 keepdims=True)
    probs_m = jnp.where(eidx == i1, -jnp.inf, probs)
    m2 = jnp.max(probs_m, axis=1, keepdims=True)
    i2 = jnp.min(jnp.where(probs_m == m2, eidx, n_experts), axis=1,
                 keepdims=True)
    ssum = m1 + m2
    g1 = m1 / ssum
    g2 = m2 / ssum

    oh = (eidx == i1).astype(f32) + (eidx == i2).astype(f32)

    me = jnp.sum(probs, axis=0, keepdims=True) / n_tokens
    ce = jnp.sum(oh, axis=0, keepdims=True) / n_tokens
    aux_ref[...] = ((n_experts / k_top) * jnp.sum(me * ce)).reshape(1, 1)

    blk = 256
    cnt_rows = []
    for b in range(n_tokens // blk):
        r = lax.broadcasted_iota(jnp.int32, (blk, n_tokens), 0) + (b * blk)
        c = lax.broadcasted_iota(jnp.int32, (blk, n_tokens), 1)
        trib = (c < r).astype(f32)
        cnt_rows.append(jnp.dot(trib, oh, preferred_element_type=f32))
    cnt = jnp.concatenate(cnt_rows, axis=0)

    p1 = jnp.sum(jnp.where(eidx == i1, cnt, 0.0), axis=1, keepdims=True)
    p2 = jnp.sum(jnp.where(eidx == i2, cnt, 0.0), axis=1, keepdims=True)
    p1i = p1.astype(jnp.int32)
    p2i = p2.astype(jnp.int32)
    v1 = p1i < capacity
    v2 = p2i < capacity
    slot1 = jnp.where(v1, i1 * capacity + p1i, 0)
    slot2 = jnp.where(v2, i2 * capacity + p2i, 0)
    slots_ref[...] = jnp.concatenate([slot1, slot2], axis=1)
    gates_ref[...] = jnp.concatenate(
        [jnp.where(v1, g1, 0.0), jnp.where(v2, g2, 0.0)], axis=1)
    valid_ref[...] = jnp.concatenate(
        [v1.astype(jnp.int32), v2.astype(jnp.int32)], axis=1)


def _router(x_flat, Wr, capacity, k_top):
    n_tokens, _ = x_flat.shape
    n_experts = Wr.shape[1]
    body = functools.partial(_router_body, n_tokens, n_experts, capacity,
                             k_top)
    return pl.pallas_call(
        body,
        out_shape=[
            jax.ShapeDtypeStruct((n_tokens, k_top), jnp.int32),
            jax.ShapeDtypeStruct((n_tokens, k_top), jnp.float32),
            jax.ShapeDtypeStruct((n_tokens, k_top), jnp.int32),
            jax.ShapeDtypeStruct((1, 1), jnp.float32),
        ],
    )(x_flat, Wr)


def _make_scatter_src(n_pairs, n_slots):
    mesh = plsc.VectorSubcoreMesh(core_axis_name="c", subcore_axis_name="s")

    @functools.partial(
        pl.kernel,
        mesh=mesh,
        out_type=jax.ShapeDtypeStruct((n_slots,), jnp.int32),
        scratch_types=[
            pltpu.VMEM((n_pairs,), jnp.int32),
            pltpu.VMEM((n_pairs,), jnp.int32),
            pltpu.VMEM((n_slots,), jnp.int32),
        ],
        compiler_params=pltpu.CompilerParams(needs_layout_passes=False),
    )
    def k(slots_hbm, valid_hbm, out_hbm, slots_v, valid_v, src_v):
        wid = lax.axis_index("c") * _NS + lax.axis_index("s")

        @pl.when(wid == 0)
        def _():
            pltpu.sync_copy(slots_hbm, slots_v)
            pltpu.sync_copy(valid_hbm, valid_v)

            def zbody(i, carry):
                src_v[pl.ds(i * 16, 16)] = jnp.zeros((16,), jnp.int32)
                return carry

            lax.fori_loop(0, n_slots // 16, zbody, 0)

            def sbody(j, carry):
                sl = slots_v[pl.ds(j * 16, 16)]
                vm = valid_v[pl.ds(j * 16, 16)] > 0
                pair_ids = j * 16 + lax.broadcasted_iota(jnp.int32, (16,), 0)
                toks = lax.shift_right_logical(pair_ids, 1)
                plsc.store_scatter(src_v, [sl], toks, mask=vm)
                return carry

            lax.fori_loop(0, n_pairs // 16, sbody, 0)
            pltpu.sync_copy(src_v, out_hbm)

    return k


def _make_gather(n_table, d, n_rows, chunk):
    assert n_rows % (_NW * chunk) == 0
    rpw = n_rows // _NW
    mesh = plsc.VectorSubcoreMesh(core_axis_name="c", subcore_axis_name="s")

    @functools.partial(
        pl.kernel,
        mesh=mesh,
        out_type=jax.ShapeDtypeStruct((n_rows, d), jnp.float32),
        scratch_types=[
            pltpu.VMEM((chunk,), jnp.int32),
            pltpu.VMEM((chunk, d), jnp.float32),
            pltpu.SemaphoreType.DMA,
        ],
        compiler_params=pltpu.CompilerParams(needs_layout_passes=False),
    )
    def k(table_hbm, idx_hbm, out_hbm, idx_v, rows_v, sem):
        wid = lax.axis_index("c") * _NS + lax.axis_index("s")
        base = wid * rpw

        def body(i, carry):
            off = base + i * chunk
            pltpu.sync_copy(idx_hbm.at[pl.ds(off, chunk)], idx_v)
            pltpu.async_copy(table_hbm.at[idx_v], rows_v, sem).wait()
            pltpu.sync_copy(rows_v, out_hbm.at[pl.ds(off, chunk), :])
            return carry

        lax.fori_loop(0, rpw // chunk, body, 0)

    return k


def _ffn_body(bf, x_ref, w1_ref, b1_ref, w2_ref, b2_ref, out_ref):
    j = pl.program_id(1)
    h = jnp.dot(x_ref[...], w1_ref[0], preferred_element_type=jnp.float32)
    h = jax.nn.gelu(h + b1_ref[0])
    contrib = jnp.dot(h, w2_ref[0], preferred_element_type=jnp.float32)

    @pl.when(j == 0)
    def _init():
        out_ref[...] = contrib + b2_ref[0]

    @pl.when(j != 0)
    def _acc():
        out_ref[...] = out_ref[...] + contrib


def _ffn(xe, w1, b1, w2, b2, capacity):
    n_experts, d_model, d_ff = w1.shape
    bf = 512
    body = functools.partial(_ffn_body, bf)
    b1r = b1.reshape(n_experts, 1, d_ff)
    b2r = b2.reshape(n_experts, 1, d_model)
    return pl.pallas_call(
        body,
        grid=(n_experts, d_ff // bf),
        in_specs=[
            pl.BlockSpec((capacity, d_model), lambda e, j: (e, 0)),
            pl.BlockSpec((1, d_model, bf), lambda e, j: (e, 0, j)),
            pl.BlockSpec((1, 1, bf), lambda e, j: (e, 0, j)),
            pl.BlockSpec((1, bf, d_model), lambda e, j: (e, j, 0)),
            pl.BlockSpec((1, 1, d_model), lambda e, j: (e, 0, 0)),
        ],
        out_specs=pl.BlockSpec((capacity, d_model), lambda e, j: (e, 0)),
        out_shape=jax.ShapeDtypeStruct((n_experts * capacity, d_model),
                                       jnp.float32),
        compiler_params=pltpu.CompilerParams(
            dimension_semantics=("parallel", "arbitrary")),
    )(xe, w1, b1r, w2, b2r)


def _combine_body(g_ref, w_ref, out_ref):
    g = g_ref[...]
    w = w_ref[...]
    out_ref[...] = g[:, 0, :] * w[:, 0:1] + g[:, 1, :] * w[:, 1:2]


def _combine(gathered3, gates2):
    n_tokens, k_top, d_model = gathered3.shape
    blk = 256
    return pl.pallas_call(
        _combine_body,
        grid=(n_tokens // blk,),
        in_specs=[
            pl.BlockSpec((blk, k_top, d_model), lambda i: (i, 0, 0)),
            pl.BlockSpec((blk, k_top), lambda i: (i, 0)),
        ],
        out_specs=pl.BlockSpec((blk, d_model), lambda i: (i, 0)),
        out_shape=jax.ShapeDtypeStruct((n_tokens, d_model), jnp.float32),
    )(gathered3, gates2)


def kernel(x, Wr, w1, b1, w2, b2):
    B, T, d_model = x.shape
    n_experts = Wr.shape[1]
    k_top = 2
    n_tokens = B * T
    n_pairs = n_tokens * k_top
    capacity = math.ceil(1.25 * n_pairs / n_experts)
    n_slots = n_experts * capacity

    x_flat = x.reshape(n_tokens, d_model)
    slots2, gates2, valid2, aux = _router(x_flat, Wr, capacity, k_top)
    slots_flat = slots2.reshape(-1)
    valid_flat = valid2.reshape(-1)

    src_tok = _make_scatter_src(n_pairs, n_slots)(slots_flat, valid_flat)
    xe = _make_gather(n_tokens, d_model, n_slots, 32)(x_flat, src_tok)
    out_e = _ffn(xe, w1, b1, w2, b2, capacity)
    rows = _make_gather(n_slots, d_model, n_pairs, 32)(out_e, slots_flat)
    y_flat = _combine(rows.reshape(n_tokens, k_top, d_model), gates2)
    return y_flat.reshape(B, T, d_model), aux.reshape(())


# speedup vs baseline: 1.7644x; 1.2327x over previous
import functools
import math

import jax
import jax.numpy as jnp
from jax import lax
from jax.experimental import pallas as pl
from jax.experimental.pallas import tpu as pltpu
from jax.experimental.pallas import tpu_sc as plsc

_NC = 2
_NS = 16
_NW = _NC * _NS


def _router_body(n_tokens, n_experts, capacity, k_top,
                 x_ref, wr_ref, slots_ref, gates_ref, valid_ref, aux_ref):
    f32 = jnp.float32
    xb = x_ref[...]
    wr = wr_ref[...]
    logits = jnp.dot(xb, wr, preferred_element_type=f32)
    m = jnp.max(logits, axis=1, keepdims=True)
    ex = jnp.exp(logits - m)
    probs = ex / jnp.sum(ex, axis=1, keepdims=True)

    eidx = lax.broadcasted_iota(jnp.int32, (n_tokens, n_experts), 1)
    m1 = jnp.max(probs, axis=1, keepdims=True)
    i1 = jnp.min(jnp.where(probs == m1, eidx, n_experts), axis=1,
                 keepdims=True)
    probs_m = jnp.where(eidx == i1, -jnp.inf, probs)
    m2 = jnp.max(probs_m, axis=1, keepdims=True)
    i2 = jnp.min(jnp.where(probs_m == m2, eidx, n_experts), axis=1,
                 keepdims=True)
    ssum = m1 + m2
    g1 = m1 / ssum
    g2 = m2 / ssum

    oh = (eidx == i1).astype(f32) + (eidx == i2).astype(f32)

    me = jnp.sum(probs, axis=0, keepdims=True) / n_tokens
    ce = jnp.sum(oh, axis=0, keepdims=True) / n_tokens
    aux_ref[...] = ((n_experts / k_top) * jnp.sum(me * ce)).reshape(1, 1)

    blk = 256
    cnt_rows = []
    for b in range(n_tokens // blk):
        r = lax.broadcasted_iota(jnp.int32, (blk, n_tokens), 0) + (b * blk)
        c = lax.broadcasted_iota(jnp.int32, (blk, n_tokens), 1)
        trib = (c < r).astype(f32)
        cnt_rows.append(jnp.dot(trib, oh, preferred_element_type=f32))
    cnt = jnp.concatenate(cnt_rows, axis=0)

    p1 = jnp.sum(jnp.where(eidx == i1, cnt, 0.0), axis=1, keepdims=True)
    p2 = jnp.sum(jnp.where(eidx == i2, cnt, 0.0), axis=1, keepdims=True)
    p1i = p1.astype(jnp.int32)
    p2i = p2.astype(jnp.int32)
    v1 = p1i < capacity
    v2 = p2i < capacity
    slot1 = jnp.where(v1, i1 * capacity + p1i, 0)
    slot2 = jnp.where(v2, i2 * capacity + p2i, 0)
    slots_ref[...] = jnp.concatenate([slot1, slot2], axis=1)
    gates_ref[...] = jnp.concatenate(
        [jnp.where(v1, g1, 0.0), jnp.where(v2, g2, 0.0)], axis=1)
    valid_ref[...] = jnp.concatenate(
        [v1.astype(jnp.int32), v2.astype(jnp.int32)], axis=1)


def _router(x_flat, Wr, capacity, k_top):
    n_tokens, _ = x_flat.shape
    n_experts = Wr.shape[1]
    body = functools.partial(_router_body, n_tokens, n_experts, capacity,
                             k_top)
    return pl.pallas_call(
        body,
        out_shape=[
            jax.ShapeDtypeStruct((n_tokens, k_top), jnp.int32),
            jax.ShapeDtypeStruct((n_tokens, k_top), jnp.float32),
            jax.ShapeDtypeStruct((n_tokens, k_top), jnp.int32),
            jax.ShapeDtypeStruct((1, 1), jnp.float32),
        ],
    )(x_flat, Wr)


def _make_scatter_src(n_pairs, n_slots):
    mesh = plsc.VectorSubcoreMesh(core_axis_name="c", subcore_axis_name="s")

    @functools.partial(
        pl.kernel,
        mesh=mesh,
        out_type=jax.ShapeDtypeStruct((n_slots,), jnp.int32),
        scratch_types=[
            pltpu.VMEM((n_pairs,), jnp.int32),
            pltpu.VMEM((n_pairs,), jnp.int32),
            pltpu.VMEM((n_slots,), jnp.int32),
        ],
        compiler_params=pltpu.CompilerParams(needs_layout_passes=False),
    )
    def k(slots_hbm, valid_hbm, out_hbm, slots_v, valid_v, src_v):
        wid = lax.axis_index("c") * _NS + lax.axis_index("s")

        @pl.when(wid == 0)
        def _():
            pltpu.sync_copy(slots_hbm, slots_v)
            pltpu.sync_copy(valid_hbm, valid_v)

            def zbody(i, carry):
                sl_ids = i * 16 + lax.broadcasted_iota(jnp.int32, (16,), 0)
                src_v[pl.ds(i * 16, 16)] = lax.rem(sl_ids, n_pairs // 2)
                return carry

            lax.fori_loop(0, n_slots // 16, zbody, 0)

            def sbody(j, carry):
                sl = slots_v[pl.ds(j * 16, 16)]
                vm = valid_v[pl.ds(j * 16, 16)] > 0
                pair_ids = j * 16 + lax.broadcasted_iota(jnp.int32, (16,), 0)
                toks = lax.shift_right_logical(pair_ids, 1)
                plsc.store_scatter(src_v, [sl], toks, mask=vm)
                return carry

            lax.fori_loop(0, n_pairs // 16, sbody, 0)
            pltpu.sync_copy(src_v, out_hbm)

    return k


def _make_gather(n_table, d, n_rows, n_chunks):
    rpw = n_rows // _NW
    chunk = rpw // n_chunks
    assert rpw % n_chunks == 0 and chunk % 8 == 0
    nbuf = min(3, n_chunks)
    mesh = plsc.VectorSubcoreMesh(core_axis_name="c", subcore_axis_name="s")

    @functools.partial(
        pl.kernel,
        mesh=mesh,
        out_type=jax.ShapeDtypeStruct((n_rows, d), jnp.float32),
        scratch_types=[
            pltpu.VMEM((rpw,), jnp.int32),
            [pltpu.VMEM((chunk, d), jnp.float32) for _ in range(nbuf)],
            [pltpu.SemaphoreType.DMA for _ in range(nbuf)],
            [pltpu.SemaphoreType.DMA for _ in range(nbuf)],
        ],
        compiler_params=pltpu.CompilerParams(needs_layout_passes=False),
    )
    def k(table_hbm, idx_hbm, out_hbm, idx_v, bufs, gsems, wsems):
        wid = lax.axis_index("c") * _NS + lax.axis_index("s")
        base = wid * rpw
        pltpu.sync_copy(idx_hbm.at[pl.ds(base, rpw)], idx_v)

        def start_gather(c):
            return pltpu.async_copy(
                table_hbm.at[idx_v.at[pl.ds(c * chunk, chunk)]],
                bufs[c % nbuf], gsems[c % nbuf])

        ghandles = [None] * n_chunks
        whandles = [None] * n_chunks
        for c in range(min(nbuf - 1, n_chunks)):
            ghandles[c] = start_gather(c)
        for c in range(n_chunks):
            pre = c + nbuf - 1
            if pre < n_chunks:
                if pre - nbuf >= 0:
                    whandles[pre - nbuf].wait()
                ghandles[pre] = start_gather(pre)
            ghandles[c].wait()
            whandles[c] = pltpu.async_copy(
                bufs[c % nbuf], out_hbm.at[pl.ds(base + c * chunk, chunk)],
                wsems[c % nbuf])
        for c in range(max(0, n_chunks - nbuf), n_chunks):
            whandles[c].wait()

    return k


def _ffn_body(bf, x_ref, w1_ref, b1_ref, w2_ref, b2_ref, out_ref):
    j = pl.program_id(1)
    h = jnp.dot(x_ref[...], w1_ref[0], preferred_element_type=jnp.float32)
    h = jax.nn.gelu(h + b1_ref[0])
    contrib = jnp.dot(h, w2_ref[0], preferred_element_type=jnp.float32)

    @pl.when(j == 0)
    def _init():
        out_ref[...] = contrib + b2_ref[0]

    @pl.when(j != 0)
    def _acc():
        out_ref[...] = out_ref[...] + contrib


def _ffn(xe, w1, b1, w2, b2, capacity):
    n_experts, d_model, d_ff = w1.shape
    bf = 512
    body = functools.partial(_ffn_body, bf)
    b1r = b1.reshape(n_experts, 1, d_ff)
    b2r = b2.reshape(n_experts, 1, d_model)
    return pl.pallas_call(
        body,
        grid=(n_experts, d_ff // bf),
        in_specs=[
            pl.BlockSpec((capacity, d_model), lambda e, j: (e, 0)),
            pl.BlockSpec((1, d_model, bf), lambda e, j: (e, 0, j)),
            pl.BlockSpec((1, 1, bf), lambda e, j: (e, 0, j)),
            pl.BlockSpec((1, bf, d_model), lambda e, j: (e, j, 0)),
            pl.BlockSpec((1, 1, d_model), lambda e, j: (e, 0, 0)),
        ],
        out_specs=pl.BlockSpec((capacity, d_model), lambda e, j: (e, 0)),
        out_shape=jax.ShapeDtypeStruct((n_experts * capacity, d_model),
                                       jnp.float32),
        compiler_params=pltpu.CompilerParams(
            dimension_semantics=("parallel", "arbitrary")),
    )(xe, w1, b1r, w2, b2r)


def _combine_body(g_ref, w_ref, out_ref):
    g = g_ref[...]
    w = w_ref[...]
    out_ref[...] = g[:, 0, :] * w[:, 0:1] + g[:, 1, :] * w[:, 1:2]


def _combine(gathered3, gates2):
    n_tokens, k_top, d_model = gathered3.shape
    blk = 256
    return pl.pallas_call(
        _combine_body,
        grid=(n_tokens // blk,),
        in_specs=[
            pl.BlockSpec((blk, k_top, d_model), lambda i: (i, 0, 0)),
            pl.BlockSpec((blk, k_top), lambda i: (i, 0)),
        ],
        out_specs=pl.BlockSpec((blk, d_model), lambda i: (i, 0)),
        out_shape=jax.ShapeDtypeStruct((n_tokens, d_model), jnp.float32),
    )(gathered3, gates2)


def kernel(x, Wr, w1, b1, w2, b2):
    B, T, d_model = x.shape
    n_experts = Wr.shape[1]
    k_top = 2
    n_tokens = B * T
    n_pairs = n_tokens * k_top
    capacity = math.ceil(1.25 * n_pairs / n_experts)
    n_slots = n_experts * capacity

    x_flat = x.reshape(n_tokens, d_model)
    slots2, gates2, valid2, aux = _router(x_flat, Wr, capacity, k_top)
    slots_flat = slots2.reshape(-1)
    valid_flat = valid2.reshape(-1)

    src_tok = _make_scatter_src(n_pairs, n_slots)(slots_flat, valid_flat)
    xe = _make_gather(n_tokens, d_model, n_slots, 4)(x_flat, src_tok)
    out_e = _ffn(xe, w1, b1, w2, b2, capacity)
    rows = _make_gather(n_slots, d_model, n_pairs, 4)(out_e, slots_flat)
    y_flat = _combine(rows.reshape(n_tokens, k_top, d_model), gates2)
    return y_flat.reshape(B, T, d_model), aux.reshape(())
